# Initial kernel scaffold; baseline (speedup 1.0000x reference)
import jax
import jax.numpy as jnp
from jax.experimental import pallas as pl


def kernel(x, edge_index, edge_weight, W, b):
    raise NotImplementedError("write your pallas kernel here")


# speedup vs baseline: 4.2599x; 4.2599x over previous
import functools

import jax
import jax.numpy as jnp
from jax import lax
from jax.experimental import pallas as pl
from jax.experimental.pallas import tpu as pltpu
from jax.experimental.pallas import tpu_sc as plsc

NC = 2
NS = 16
NW = NC * NS
CHUNK = 128
LANES = 16


def _sc_accumulate(src2d, dst2d, ew, x_halves, v_pad, n_chunks):
    ch = x_halves.shape[2]
    epw = n_chunks * CHUNK
    rows_per_tile = v_pad // NS
    mesh = plsc.VectorSubcoreMesh(core_axis_name="c", subcore_axis_name="s")

    @functools.partial(
        pl.kernel,
        out_type=[
            jax.ShapeDtypeStruct((NC, v_pad, ch), jnp.float32),
            jax.ShapeDtypeStruct((v_pad, LANES), jnp.float32),
        ],
        mesh=mesh,
        compiler_params=pltpu.CompilerParams(use_tc_tiling_on_sc=False),
        scratch_types=[
            pltpu.VMEM((n_chunks, CHUNK), jnp.int32),
            pltpu.VMEM((n_chunks, CHUNK), jnp.int32),
            pltpu.VMEM((epw,), jnp.float32),
            pltpu.VMEM((CHUNK, ch), jnp.float32),
            pltpu.VMEM((CHUNK, LANES), jnp.float32),
            pltpu.VMEM_SHARED((v_pad, ch), jnp.float32),
            pltpu.VMEM_SHARED((v_pad, LANES), jnp.float32),
            pltpu.SemaphoreType.DMA,
        ],
    )
    def sc_kern(src_hbm, dst_hbm, ew_hbm, x_hbm, g_out, s_out,
                src_v, dst_v, ew_v, rows_v, wrow_v, g_sh, s_sh, sem):
        cid = lax.axis_index("c")
        sid = lax.axis_index("s")

        zeros16 = jnp.zeros((LANES,), jnp.float32)

        def zero_body(i, _):
            for j in range(ch // LANES):
                rows_v[i, pl.ds(j * LANES, LANES)] = zeros16
            wrow_v[i, :] = zeros16
            return 0
        lax.fori_loop(0, CHUNK, zero_body, 0)

        r0 = sid * rows_per_tile
        for k in range(rows_per_tile // CHUNK):
            pltpu.sync_copy(rows_v, g_sh.at[pl.ds(r0 + k * CHUNK, CHUNK)])
            pltpu.sync_copy(wrow_v, s_sh.at[pl.ds(r0 + k * CHUNK, CHUNK)])
        plsc.subcore_barrier()

        pltpu.sync_copy(src_hbm.at[pl.ds(sid * n_chunks, n_chunks)], src_v)
        pltpu.sync_copy(dst_hbm.at[pl.ds(sid * n_chunks, n_chunks)], dst_v)
        pltpu.sync_copy(ew_hbm.at[pl.ds(sid * epw, epw)], ew_v)

        def chunk_body(c, _):
            pltpu.async_copy(x_hbm.at[cid].at[src_v.at[c]], rows_v, sem).wait()

            base = c * CHUNK

            def group_body(g, _):
                w16 = ew_v[pl.ds(base + g * LANES, LANES)]
                for l in range(LANES):
                    i = g * LANES + l
                    w = jnp.broadcast_to(w16[l], (LANES,))
                    for j in range(ch // LANES):
                        rows_v[i, pl.ds(j * LANES, LANES)] = (
                            rows_v[i, pl.ds(j * LANES, LANES)] * w)
                    wrow_v[i, :] = w
                return 0
            lax.fori_loop(0, CHUNK // LANES, group_body, 0)

            pltpu.sync_copy(rows_v, g_sh.at[dst_v.at[c]], add=True)

            @pl.when(cid == 0)
            def _():
                pltpu.sync_copy(wrow_v, s_sh.at[dst_v.at[c]], add=True)
            return 0
        lax.fori_loop(0, n_chunks, chunk_body, 0)

        plsc.subcore_barrier()

        pltpu.sync_copy(g_sh.at[pl.ds(r0, rows_per_tile)],
                        g_out.at[cid, pl.ds(r0, rows_per_tile)])

        @pl.when(cid == 0)
        def _():
            pltpu.sync_copy(s_sh.at[pl.ds(r0, rows_per_tile)],
                            s_out.at[pl.ds(r0, rows_per_tile)])

    return sc_kern(src2d, dst2d, ew, x_halves)


def _tc_combine(x_pad, g_part, s_part, A, W2, b, v_pad):
    C = x_pad.shape[1]
    ch = g_part.shape[2]
    F = A.shape[1]
    VB = 1024
    grid = (v_pad // VB,)

    def body(x_ref, g_ref, s_ref, a_ref, w2_ref, b_ref, o_ref):
        s = s_ref[:, 0:1]
        sx = x_ref[...] * s
        o_ref[...] = (
            jnp.dot(sx, a_ref[...], preferred_element_type=jnp.float32)
            + jnp.dot(g_ref[0], w2_ref[0], preferred_element_type=jnp.float32)
            + jnp.dot(g_ref[1], w2_ref[1], preferred_element_type=jnp.float32)
            + s * b_ref[...])

    return pl.pallas_call(
        body,
        grid=grid,
        in_specs=[
            pl.BlockSpec((VB, C), lambda i: (i, 0)),
            pl.BlockSpec((NC, VB, ch), lambda i: (0, i, 0)),
            pl.BlockSpec((VB, LANES), lambda i: (i, 0)),
            pl.BlockSpec((C, F), lambda i: (0, 0)),
            pl.BlockSpec((NC, ch, F), lambda i: (0, 0, 0)),
            pl.BlockSpec((1, F), lambda i: (0, 0)),
        ],
        out_specs=pl.BlockSpec((VB, F), lambda i: (i, 0)),
        out_shape=jax.ShapeDtypeStruct((v_pad, F), jnp.float32),
    )(x_pad, g_part, s_part, A, W2, b.reshape(1, F))


def kernel(x, edge_index, edge_weight, W, b):
    V, C = x.shape
    E = edge_index.shape[1]
    F = W.shape[1]
    ch = C // NC

    epw = -(-E // (NS * 8 * CHUNK)) * (8 * CHUNK)
    e_pad = epw * NS
    n_chunks = epw // CHUNK
    v_pad = -(-V // (NS * CHUNK)) * (NS * CHUNK)

    dst = edge_index[0]
    src = edge_index[1]
    pad = e_pad - E
    pad_idx = (jnp.arange(pad, dtype=jnp.int32) * 13) % V
    src_p = jnp.concatenate([src, pad_idx]).reshape(NS * n_chunks, CHUNK)
    dst_p = jnp.concatenate([dst, pad_idx]).reshape(NS * n_chunks, CHUNK)
    ew_p = jnp.concatenate([edge_weight, jnp.zeros((pad,), jnp.float32)])
    x_pad = jnp.pad(x, ((0, v_pad - V), (0, 0)))
    x_halves = jnp.stack([x_pad[:, :ch], x_pad[:, ch:]])

    g_part, s_part = _sc_accumulate(src_p, dst_p, ew_p, x_halves, v_pad,
                                    n_chunks)

    W1 = W[:C]
    W2 = W[C:]
    w2_halves = jnp.stack([W2[:ch], W2[ch:]])
    out = _tc_combine(x_pad, g_part, s_part, W1 - W2, w2_halves, b, v_pad)
    return out[:V]


# speedup vs baseline: 6.1575x; 1.4455x over previous
import functools

import jax
import jax.numpy as jnp
from jax import lax
from jax.experimental import pallas as pl
from jax.experimental.pallas import tpu as pltpu
from jax.experimental.pallas import tpu_sc as plsc

NC = 2
NS = 16
NW = NC * NS
CHUNK = 128
LANES = 16


def _sc_accumulate(src2d, dst2d, ew, x_halves, v_pad, n_chunks):
    ch = x_halves.shape[2]
    epw = n_chunks * CHUNK
    rows_per_tile = v_pad // NS
    mesh = plsc.VectorSubcoreMesh(core_axis_name="c", subcore_axis_name="s")

    @functools.partial(
        pl.kernel,
        out_type=[
            jax.ShapeDtypeStruct((NC, v_pad, ch), jnp.float32),
            jax.ShapeDtypeStruct((v_pad, LANES), jnp.float32),
        ],
        mesh=mesh,
        compiler_params=pltpu.CompilerParams(use_tc_tiling_on_sc=False),
        scratch_types=[
            pltpu.VMEM((n_chunks, CHUNK), jnp.int32),
            pltpu.VMEM((n_chunks, CHUNK), jnp.int32),
            pltpu.VMEM((epw,), jnp.float32),
            pltpu.VMEM((CHUNK, ch), jnp.float32),
            pltpu.VMEM((CHUNK, LANES), jnp.float32),
            pltpu.VMEM_SHARED((v_pad, ch), jnp.float32),
            pltpu.VMEM_SHARED((v_pad, LANES), jnp.float32),
            pltpu.SemaphoreType.DMA,
        ],
    )
    def sc_kern(src_hbm, dst_hbm, ew_hbm, x_hbm, g_out, s_out,
                src_v, dst_v, ew_v, rows_v, wrow_v, g_sh, s_sh, sem):
        cid = lax.axis_index("c")
        sid = lax.axis_index("s")

        zeros16 = jnp.zeros((LANES,), jnp.float32)

        def zero_body(i, _):
            for j in range(ch // LANES):
                rows_v[i, pl.ds(j * LANES, LANES)] = zeros16
            wrow_v[i, :] = zeros16
            return 0
        lax.fori_loop(0, CHUNK, zero_body, 0)

        r0 = sid * rows_per_tile
        for k in range(rows_per_tile // CHUNK):
            pltpu.sync_copy(rows_v, g_sh.at[pl.ds(r0 + k * CHUNK, CHUNK)])
            pltpu.sync_copy(wrow_v, s_sh.at[pl.ds(r0 + k * CHUNK, CHUNK)])
        plsc.subcore_barrier()

        pltpu.sync_copy(src_hbm.at[pl.ds(sid * n_chunks, n_chunks)], src_v)
        pltpu.sync_copy(dst_hbm.at[pl.ds(sid * n_chunks, n_chunks)], dst_v)
        pltpu.sync_copy(ew_hbm.at[pl.ds(sid * epw, epw)], ew_v)

        def chunk_body(c, _):
            pltpu.async_copy(x_hbm.at[cid].at[src_v.at[c]], rows_v, sem).wait()

            base = c * CHUNK

            def group_body(g, _):
                w16 = ew_v[pl.ds(base + g * LANES, LANES)]
                for l in range(LANES):
                    i = g * LANES + l
                    w = jnp.broadcast_to(w16[l], (LANES,))
                    vals = [rows_v[i, pl.ds(j * LANES, LANES)]
                            for j in range(ch // LANES)]
                    scaled = [v * w for v in vals]
                    for j in range(ch // LANES):
                        rows_v[i, pl.ds(j * LANES, LANES)] = scaled[j]
                    wrow_v[i, :] = w
                return 0
            lax.fori_loop(0, CHUNK // LANES, group_body, 0)

            pltpu.sync_copy(rows_v, g_sh.at[dst_v.at[c]], add=True)

            @pl.when(cid == 0)
            def _():
                pltpu.sync_copy(wrow_v, s_sh.at[dst_v.at[c]], add=True)
            return 0
        lax.fori_loop(0, n_chunks, chunk_body, 0)

        plsc.subcore_barrier()

        pltpu.sync_copy(g_sh.at[pl.ds(r0, rows_per_tile)],
                        g_out.at[cid, pl.ds(r0, rows_per_tile)])

        @pl.when(cid == 0)
        def _():
            pltpu.sync_copy(s_sh.at[pl.ds(r0, rows_per_tile)],
                            s_out.at[pl.ds(r0, rows_per_tile)])

    return sc_kern(src2d, dst2d, ew, x_halves)


def _tc_combine(x_pad, g_part, s_part, A, W2, b, v_pad):
    C = x_pad.shape[1]
    ch = g_part.shape[2]
    F = A.shape[1]
    VB = 1024
    grid = (v_pad // VB,)

    def body(x_ref, g_ref, s_ref, a_ref, w2_ref, b_ref, o_ref):
        s = s_ref[:, 0:1]
        sx = x_ref[...] * s
        o_ref[...] = (
            jnp.dot(sx, a_ref[...], preferred_element_type=jnp.float32)
            + jnp.dot(g_ref[0], w2_ref[0], preferred_element_type=jnp.float32)
            + jnp.dot(g_ref[1], w2_ref[1], preferred_element_type=jnp.float32)
            + s * b_ref[...])

    return pl.pallas_call(
        body,
        grid=grid,
        in_specs=[
            pl.BlockSpec((VB, C), lambda i: (i, 0)),
            pl.BlockSpec((NC, VB, ch), lambda i: (0, i, 0)),
            pl.BlockSpec((VB, LANES), lambda i: (i, 0)),
            pl.BlockSpec((C, F), lambda i: (0, 0)),
            pl.BlockSpec((NC, ch, F), lambda i: (0, 0, 0)),
            pl.BlockSpec((1, F), lambda i: (0, 0)),
        ],
        out_specs=pl.BlockSpec((VB, F), lambda i: (i, 0)),
        out_shape=jax.ShapeDtypeStruct((v_pad, F), jnp.float32),
    )(x_pad, g_part, s_part, A, W2, b.reshape(1, F))


def kernel(x, edge_index, edge_weight, W, b):
    V, C = x.shape
    E = edge_index.shape[1]
    F = W.shape[1]
    ch = C // NC

    epw = -(-E // (NS * 8 * CHUNK)) * (8 * CHUNK)
    e_pad = epw * NS
    n_chunks = epw // CHUNK
    v_pad = -(-V // (NS * CHUNK)) * (NS * CHUNK)

    dst = edge_index[0]
    src = edge_index[1]
    pad = e_pad - E
    pad_idx = (jnp.arange(pad, dtype=jnp.int32) * 13) % V
    src_p = jnp.concatenate([src, pad_idx]).reshape(NS * n_chunks, CHUNK)
    dst_p = jnp.concatenate([dst, pad_idx]).reshape(NS * n_chunks, CHUNK)
    ew_p = jnp.concatenate([edge_weight, jnp.zeros((pad,), jnp.float32)])
    x_pad = jnp.pad(x, ((0, v_pad - V), (0, 0)))
    x_halves = jnp.stack([x_pad[:, :ch], x_pad[:, ch:]])

    g_part, s_part = _sc_accumulate(src_p, dst_p, ew_p, x_halves, v_pad,
                                    n_chunks)

    W1 = W[:C]
    W2 = W[C:]
    w2_halves = jnp.stack([W2[:ch], W2[ch:]])
    out = _tc_combine(x_pad, g_part, s_part, W1 - W2, w2_halves, b, v_pad)
    return out[:V]


# speedup vs baseline: 9.2320x; 1.4993x over previous
import functools

import jax
import jax.numpy as jnp
from jax import lax
from jax.experimental import pallas as pl
from jax.experimental.pallas import tpu as pltpu
from jax.experimental.pallas import tpu_sc as plsc

NC = 2
NS = 16
NW = NC * NS
CHUNK = 128
LANES = 16
NPHASE = 2


def _sc_accumulate(src2d, dst2d, ew, x_halves, v_pad, n_chunks):
    ch = x_halves.shape[2]
    epw = n_chunks * CHUNK
    rows_per_tile = v_pad // NS
    mesh = plsc.VectorSubcoreMesh(core_axis_name="c", subcore_axis_name="s")

    @functools.partial(
        pl.kernel,
        out_type=[
            jax.ShapeDtypeStruct((NC, v_pad, ch), jnp.float32),
            jax.ShapeDtypeStruct((v_pad, LANES), jnp.float32),
        ],
        mesh=mesh,
        compiler_params=pltpu.CompilerParams(use_tc_tiling_on_sc=False),
        scratch_types=[
            pltpu.VMEM((n_chunks // NPHASE, CHUNK), jnp.int32),
            pltpu.VMEM((n_chunks // NPHASE, CHUNK), jnp.int32),
            pltpu.VMEM((epw // NPHASE,), jnp.float32),
            pltpu.VMEM((CHUNK, ch), jnp.float32),
            pltpu.VMEM((CHUNK, ch), jnp.float32),
            pltpu.VMEM((CHUNK, LANES), jnp.float32),
            pltpu.VMEM((CHUNK, LANES), jnp.float32),
            pltpu.VMEM_SHARED((v_pad, ch), jnp.float32),
            pltpu.VMEM_SHARED((v_pad, LANES), jnp.float32),
            pltpu.SemaphoreType.DMA,
            pltpu.SemaphoreType.DMA,
            pltpu.SemaphoreType.DMA,
            pltpu.SemaphoreType.DMA,
            pltpu.SemaphoreType.DMA,
            pltpu.SemaphoreType.DMA,
        ],
    )
    def sc_kern(src_hbm, dst_hbm, ew_hbm, x_hbm, g_out, s_out,
                src_v, dst_v, ew_v, rows0, rows1, wrow0, wrow1, g_sh, s_sh,
                gsem0, gsem1, ssem0, ssem1, tsem0, tsem1):
        cid = lax.axis_index("c")
        sid = lax.axis_index("s")

        zeros16 = jnp.zeros((LANES,), jnp.float32)

        def zero_body(i, _):
            for j in range(ch // LANES):
                rows0[i, pl.ds(j * LANES, LANES)] = zeros16
            wrow0[i, :] = zeros16
            return 0
        lax.fori_loop(0, CHUNK, zero_body, 0)

        r0 = sid * rows_per_tile
        for k in range(rows_per_tile // CHUNK):
            pltpu.sync_copy(rows0, g_sh.at[pl.ds(r0 + k * CHUNK, CHUNK)])
            pltpu.sync_copy(wrow0, s_sh.at[pl.ds(r0 + k * CHUNK, CHUNK)])
        plsc.subcore_barrier()

        def scale(rows, wrow, c):
            base = c * CHUNK

            def group_body(g, _):
                w16 = ew_v[pl.ds(base + g * LANES, LANES)]
                for l in range(LANES):
                    i = g * LANES + l
                    w = jnp.broadcast_to(w16[l], (LANES,))
                    vals = [rows[i, pl.ds(j * LANES, LANES)]
                            for j in range(ch // LANES)]
                    scaled = [v * w for v in vals]
                    for j in range(ch // LANES):
                        rows[i, pl.ds(j * LANES, LANES)] = scaled[j]
                    wrow[i, :] = w
                return 0
            lax.fori_loop(0, CHUNK // LANES, group_body, 0)

        def gather(rows, c, gsem):
            pltpu.async_copy(x_hbm.at[cid].at[src_v.at[c]], rows, gsem)

        def gather_wait(rows, c, gsem):
            pltpu.make_async_copy(x_hbm.at[cid].at[src_v.at[c]], rows,
                                  gsem).wait()

        def scatter(rows, wrow, c, ssem, tsem):
            pltpu.async_copy(rows, g_sh.at[dst_v.at[c]], ssem, add=True)

            @pl.when(cid == 0)
            def _():
                pltpu.async_copy(wrow, s_sh.at[dst_v.at[c]], tsem, add=True)

        def scatter_wait(rows, wrow, c, ssem, tsem):
            pltpu.make_async_copy(rows, g_sh.at[dst_v.at[c]], ssem).wait()

            @pl.when(cid == 0)
            def _():
                pltpu.make_async_copy(wrow, s_sh.at[dst_v.at[c]], tsem).wait()

        n_ph = n_chunks // NPHASE
        nsup = n_ph // 2
        for ph in range(NPHASE):
            pltpu.sync_copy(
                src_hbm.at[pl.ds(sid * n_chunks + ph * n_ph, n_ph)], src_v)
            pltpu.sync_copy(
                dst_hbm.at[pl.ds(sid * n_chunks + ph * n_ph, n_ph)], dst_v)
            pltpu.sync_copy(
                ew_hbm.at[pl.ds(sid * epw + ph * n_ph * CHUNK,
                                n_ph * CHUNK)], ew_v)

            gather(rows0, 0, gsem0)

            def super_body(k, _):
                c0 = 2 * k
                gather_wait(rows0, c0, gsem0)

                @pl.when(k > 0)
                def _():
                    scatter_wait(rows1, wrow1, c0 - 1, ssem1, tsem1)
                gather(rows1, c0 + 1, gsem1)
                scale(rows0, wrow0, c0)
                scatter(rows0, wrow0, c0, ssem0, tsem0)

                gather_wait(rows1, c0 + 1, gsem1)
                scatter_wait(rows0, wrow0, c0, ssem0, tsem0)

                @pl.when(k < nsup - 1)
                def _():
                    gather(rows0, c0 + 2, gsem0)
                scale(rows1, wrow1, c0 + 1)
                scatter(rows1, wrow1, c0 + 1, ssem1, tsem1)
                return 0
            lax.fori_loop(0, nsup, super_body, 0)
            scatter_wait(rows1, wrow1, n_ph - 1, ssem1, tsem1)

        plsc.subcore_barrier()

        pltpu.sync_copy(g_sh.at[pl.ds(r0, rows_per_tile)],
                        g_out.at[cid, pl.ds(r0, rows_per_tile)])

        @pl.when(cid == 0)
        def _():
            pltpu.sync_copy(s_sh.at[pl.ds(r0, rows_per_tile)],
                            s_out.at[pl.ds(r0, rows_per_tile)])

    return sc_kern(src2d, dst2d, ew, x_halves)


def _tc_combine(x_pad, g_part, s_part, A, W2, b, v_pad):
    C = x_pad.shape[1]
    ch = g_part.shape[2]
    F = A.shape[1]
    VB = 1024
    grid = (v_pad // VB,)

    def body(x_ref, g_ref, s_ref, a_ref, w2_ref, b_ref, o_ref):
        s = s_ref[:, 0:1]
        sx = x_ref[...] * s
        o_ref[...] = (
            jnp.dot(sx, a_ref[...], preferred_element_type=jnp.float32)
            + jnp.dot(g_ref[0], w2_ref[0], preferred_element_type=jnp.float32)
            + jnp.dot(g_ref[1], w2_ref[1], preferred_element_type=jnp.float32)
            + s * b_ref[...])

    return pl.pallas_call(
        body,
        grid=grid,
        in_specs=[
            pl.BlockSpec((VB, C), lambda i: (i, 0)),
            pl.BlockSpec((NC, VB, ch), lambda i: (0, i, 0)),
            pl.BlockSpec((VB, LANES), lambda i: (i, 0)),
            pl.BlockSpec((C, F), lambda i: (0, 0)),
            pl.BlockSpec((NC, ch, F), lambda i: (0, 0, 0)),
            pl.BlockSpec((1, F), lambda i: (0, 0)),
        ],
        out_specs=pl.BlockSpec((VB, F), lambda i: (i, 0)),
        out_shape=jax.ShapeDtypeStruct((v_pad, F), jnp.float32),
    )(x_pad, g_part, s_part, A, W2, b.reshape(1, F))


def kernel(x, edge_index, edge_weight, W, b):
    V, C = x.shape
    E = edge_index.shape[1]
    F = W.shape[1]
    ch = C // NC

    epw = -(-E // (NS * 8 * CHUNK)) * (8 * CHUNK)
    e_pad = epw * NS
    n_chunks = epw // CHUNK
    v_pad = -(-V // (NS * CHUNK)) * (NS * CHUNK)

    dst = edge_index[0]
    src = edge_index[1]
    pad = e_pad - E
    pad_idx = (jnp.arange(pad, dtype=jnp.int32) * 13) % V
    src_p = jnp.concatenate([src, pad_idx]).reshape(NS * n_chunks, CHUNK)
    dst_p = jnp.concatenate([dst, pad_idx]).reshape(NS * n_chunks, CHUNK)
    ew_p = jnp.concatenate([edge_weight, jnp.zeros((pad,), jnp.float32)])
    x_pad = jnp.pad(x, ((0, v_pad - V), (0, 0)))
    x_halves = jnp.stack([x_pad[:, :ch], x_pad[:, ch:]])

    g_part, s_part = _sc_accumulate(src_p, dst_p, ew_p, x_halves, v_pad,
                                    n_chunks)

    W1 = W[:C]
    W2 = W[C:]
    w2_halves = jnp.stack([W2[:ch], W2[ch:]])
    out = _tc_combine(x_pad, g_part, s_part, W1 - W2, w2_halves, b, v_pad)
    return out[:V]


# speedup vs baseline: 9.3166x; 1.0092x over previous
import functools

import jax
import jax.numpy as jnp
from jax import lax
from jax.experimental import pallas as pl
from jax.experimental.pallas import tpu as pltpu
from jax.experimental.pallas import tpu_sc as plsc

NC = 2
NS = 16
NW = NC * NS
CHUNK = 256
LANES = 16
NPHASE = 2


def _sc_accumulate(src2d, dst2d, ew, x_halves, v_pad, n_chunks):
    ch = x_halves.shape[2]
    epw = n_chunks * CHUNK
    rows_per_tile = v_pad // NS
    mesh = plsc.VectorSubcoreMesh(core_axis_name="c", subcore_axis_name="s")

    @functools.partial(
        pl.kernel,
        out_type=[
            jax.ShapeDtypeStruct((NC, v_pad, ch), jnp.float32),
            jax.ShapeDtypeStruct((v_pad, LANES), jnp.float32),
        ],
        mesh=mesh,
        compiler_params=pltpu.CompilerParams(use_tc_tiling_on_sc=False),
        scratch_types=[
            pltpu.VMEM((n_chunks // NPHASE, CHUNK), jnp.int32),
            pltpu.VMEM((n_chunks // NPHASE, CHUNK), jnp.int32),
            pltpu.VMEM((epw // NPHASE,), jnp.float32),
            pltpu.VMEM((CHUNK, ch), jnp.float32),
            pltpu.VMEM((CHUNK, ch), jnp.float32),
            pltpu.VMEM((CHUNK, LANES), jnp.float32),
            pltpu.VMEM((CHUNK, LANES), jnp.float32),
            pltpu.VMEM_SHARED((v_pad, ch), jnp.float32),
            pltpu.VMEM_SHARED((v_pad, LANES), jnp.float32),
            pltpu.SemaphoreType.DMA,
            pltpu.SemaphoreType.DMA,
            pltpu.SemaphoreType.DMA,
            pltpu.SemaphoreType.DMA,
            pltpu.SemaphoreType.DMA,
            pltpu.SemaphoreType.DMA,
        ],
    )
    def sc_kern(src_hbm, dst_hbm, ew_hbm, x_hbm, g_out, s_out,
                src_v, dst_v, ew_v, rows0, rows1, wrow0, wrow1, g_sh, s_sh,
                gsem0, gsem1, ssem0, ssem1, tsem0, tsem1):
        cid = lax.axis_index("c")
        sid = lax.axis_index("s")

        zeros16 = jnp.zeros((LANES,), jnp.float32)

        def zero_body(i, _):
            for j in range(ch // LANES):
                rows0[i, pl.ds(j * LANES, LANES)] = zeros16
            wrow0[i, :] = zeros16
            return 0
        lax.fori_loop(0, CHUNK, zero_body, 0)

        r0 = sid * rows_per_tile
        nfull = rows_per_tile // CHUNK
        for k in range(nfull):
            pltpu.sync_copy(rows0, g_sh.at[pl.ds(r0 + k * CHUNK, CHUNK)])
            pltpu.sync_copy(wrow0, s_sh.at[pl.ds(r0 + k * CHUNK, CHUNK)])
        rem = rows_per_tile - nfull * CHUNK
        if rem:
            q0 = r0 + nfull * CHUNK
            pltpu.sync_copy(rows0.at[pl.ds(0, rem)], g_sh.at[pl.ds(q0, rem)])
            pltpu.sync_copy(wrow0.at[pl.ds(0, rem)], s_sh.at[pl.ds(q0, rem)])
        plsc.subcore_barrier()

        def scale(rows, wrow, c):
            base = c * CHUNK

            def group_body(g, _):
                w16 = ew_v[pl.ds(base + g * LANES, LANES)]
                for l in range(LANES):
                    i = g * LANES + l
                    w = jnp.broadcast_to(w16[l], (LANES,))
                    vals = [rows[i, pl.ds(j * LANES, LANES)]
                            for j in range(ch // LANES)]
                    scaled = [v * w for v in vals]
                    for j in range(ch // LANES):
                        rows[i, pl.ds(j * LANES, LANES)] = scaled[j]
                    wrow[i, :] = w
                return 0
            lax.fori_loop(0, CHUNK // LANES, group_body, 0)

        def gather(rows, c, gsem):
            pltpu.async_copy(x_hbm.at[cid].at[src_v.at[c]], rows, gsem)

        def gather_wait(rows, c, gsem):
            pltpu.make_async_copy(x_hbm.at[cid].at[src_v.at[c]], rows,
                                  gsem).wait()

        def scatter(rows, wrow, c, ssem, tsem):
            pltpu.async_copy(rows, g_sh.at[dst_v.at[c]], ssem, add=True)

            @pl.when(cid == 0)
            def _():
                pltpu.async_copy(wrow, s_sh.at[dst_v.at[c]], tsem, add=True)

        def scatter_wait(rows, wrow, c, ssem, tsem):
            pltpu.make_async_copy(rows, g_sh.at[dst_v.at[c]], ssem).wait()

            @pl.when(cid == 0)
            def _():
                pltpu.make_async_copy(wrow, s_sh.at[dst_v.at[c]], tsem).wait()

        n_ph = n_chunks // NPHASE
        nsup = n_ph // 2
        for ph in range(NPHASE):
            pltpu.sync_copy(
                src_hbm.at[pl.ds(sid * n_chunks + ph * n_ph, n_ph)], src_v)
            pltpu.sync_copy(
                dst_hbm.at[pl.ds(sid * n_chunks + ph * n_ph, n_ph)], dst_v)
            pltpu.sync_copy(
                ew_hbm.at[pl.ds(sid * epw + ph * n_ph * CHUNK,
                                n_ph * CHUNK)], ew_v)

            gather(rows0, 0, gsem0)

            def super_body(k, _):
                c0 = 2 * k
                gather_wait(rows0, c0, gsem0)

                @pl.when(k > 0)
                def _():
                    scatter_wait(rows1, wrow1, c0 - 1, ssem1, tsem1)
                gather(rows1, c0 + 1, gsem1)
                scale(rows0, wrow0, c0)
                scatter(rows0, wrow0, c0, ssem0, tsem0)

                gather_wait(rows1, c0 + 1, gsem1)
                scatter_wait(rows0, wrow0, c0, ssem0, tsem0)

                @pl.when(k < nsup - 1)
                def _():
                    gather(rows0, c0 + 2, gsem0)
                scale(rows1, wrow1, c0 + 1)
                scatter(rows1, wrow1, c0 + 1, ssem1, tsem1)
                return 0
            lax.fori_loop(0, nsup, super_body, 0)
            scatter_wait(rows1, wrow1, n_ph - 1, ssem1, tsem1)

        plsc.subcore_barrier()

        pltpu.sync_copy(g_sh.at[pl.ds(r0, rows_per_tile)],
                        g_out.at[cid, pl.ds(r0, rows_per_tile)])

        @pl.when(cid == 0)
        def _():
            pltpu.sync_copy(s_sh.at[pl.ds(r0, rows_per_tile)],
                            s_out.at[pl.ds(r0, rows_per_tile)])

    return sc_kern(src2d, dst2d, ew, x_halves)


def _tc_combine(x_pad, g_part, s_part, A, W2, b, v_pad):
    C = x_pad.shape[1]
    ch = g_part.shape[2]
    F = A.shape[1]
    VB = 1024
    grid = (v_pad // VB,)

    def body(x_ref, g_ref, s_ref, a_ref, w2_ref, b_ref, o_ref):
        s = s_ref[:, 0:1]
        sx = x_ref[...] * s
        o_ref[...] = (
            jnp.dot(sx, a_ref[...], preferred_element_type=jnp.float32)
            + jnp.dot(g_ref[0], w2_ref[0], preferred_element_type=jnp.float32)
            + jnp.dot(g_ref[1], w2_ref[1], preferred_element_type=jnp.float32)
            + s * b_ref[...])

    return pl.pallas_call(
        body,
        grid=grid,
        in_specs=[
            pl.BlockSpec((VB, C), lambda i: (i, 0)),
            pl.BlockSpec((NC, VB, ch), lambda i: (0, i, 0)),
            pl.BlockSpec((VB, LANES), lambda i: (i, 0)),
            pl.BlockSpec((C, F), lambda i: (0, 0)),
            pl.BlockSpec((NC, ch, F), lambda i: (0, 0, 0)),
            pl.BlockSpec((1, F), lambda i: (0, 0)),
        ],
        out_specs=pl.BlockSpec((VB, F), lambda i: (i, 0)),
        out_shape=jax.ShapeDtypeStruct((v_pad, F), jnp.float32),
    )(x_pad, g_part, s_part, A, W2, b.reshape(1, F))


def kernel(x, edge_index, edge_weight, W, b):
    V, C = x.shape
    E = edge_index.shape[1]
    F = W.shape[1]
    ch = C // NC

    epw = -(-E // (NS * 8 * CHUNK)) * (8 * CHUNK)
    e_pad = epw * NS
    n_chunks = epw // CHUNK
    v_pad = -(-V // (NS * 128)) * (NS * 128)

    dst = edge_index[0]
    src = edge_index[1]
    pad = e_pad - E
    pad_idx = (jnp.arange(pad, dtype=jnp.int32) * 13) % V
    src_p = jnp.concatenate([src, pad_idx]).reshape(NS * n_chunks, CHUNK)
    dst_p = jnp.concatenate([dst, pad_idx]).reshape(NS * n_chunks, CHUNK)
    ew_p = jnp.concatenate([edge_weight, jnp.zeros((pad,), jnp.float32)])
    x_pad = jnp.pad(x, ((0, v_pad - V), (0, 0)))
    x_halves = jnp.stack([x_pad[:, :ch], x_pad[:, ch:]])

    g_part, s_part = _sc_accumulate(src_p, dst_p, ew_p, x_halves, v_pad,
                                    n_chunks)

    W1 = W[:C]
    W2 = W[C:]
    w2_halves = jnp.stack([W2[:ch], W2[ch:]])
    out = _tc_combine(x_pad, g_part, s_part, W1 - W2, w2_halves, b, v_pad)
    return out[:V]


# speedup vs baseline: 9.8182x; 1.0538x over previous
import functools

import jax
import jax.numpy as jnp
from jax import lax
from jax.experimental import pallas as pl
from jax.experimental.pallas import tpu as pltpu
from jax.experimental.pallas import tpu_sc as plsc

NC = 2
NS = 16
NW = NC * NS
CHUNK = 256
LANES = 16
NPHASE = 2


def _sc_accumulate(idx_p, ew, x_halves, v_pad, n_chunks):
    ch = x_halves.shape[2]
    epw = n_chunks * CHUNK
    rows_per_tile = v_pad // NS
    mesh = plsc.VectorSubcoreMesh(core_axis_name="c", subcore_axis_name="s")

    @functools.partial(
        pl.kernel,
        out_type=[
            jax.ShapeDtypeStruct((NC, v_pad, ch), jnp.float32),
            jax.ShapeDtypeStruct((v_pad, LANES), jnp.float32),
        ],
        mesh=mesh,
        compiler_params=pltpu.CompilerParams(use_tc_tiling_on_sc=False),
        scratch_types=[
            pltpu.VMEM((n_chunks // NPHASE, CHUNK), jnp.int32),
            pltpu.VMEM((n_chunks // NPHASE, CHUNK), jnp.int32),
            pltpu.VMEM((epw // NPHASE,), jnp.float32),
            pltpu.VMEM((CHUNK, ch), jnp.float32),
            pltpu.VMEM((CHUNK, ch), jnp.float32),
            pltpu.VMEM((CHUNK, LANES), jnp.float32),
            pltpu.VMEM((CHUNK, LANES), jnp.float32),
            pltpu.VMEM_SHARED((v_pad, ch), jnp.float32),
            pltpu.VMEM_SHARED((v_pad, LANES), jnp.float32),
            pltpu.SemaphoreType.DMA,
            pltpu.SemaphoreType.DMA,
            pltpu.SemaphoreType.DMA,
            pltpu.SemaphoreType.DMA,
            pltpu.SemaphoreType.DMA,
            pltpu.SemaphoreType.DMA,
        ],
    )
    def sc_kern(idx_hbm, ew_hbm, x_hbm, g_out, s_out,
                src_v, dst_v, ew_v, rows0, rows1, wrow0, wrow1, g_sh, s_sh,
                gsem0, gsem1, ssem0, ssem1, tsem0, tsem1):
        cid = lax.axis_index("c")
        sid = lax.axis_index("s")

        zeros16 = jnp.zeros((LANES,), jnp.float32)

        def zero_body(i, _):
            for j in range(ch // LANES):
                rows0[i, pl.ds(j * LANES, LANES)] = zeros16
            wrow0[i, :] = zeros16
            return 0
        lax.fori_loop(0, CHUNK, zero_body, 0)

        r0 = sid * rows_per_tile
        nfull = rows_per_tile // CHUNK
        for k in range(nfull):
            pltpu.sync_copy(rows0, g_sh.at[pl.ds(r0 + k * CHUNK, CHUNK)])
            pltpu.sync_copy(wrow0, s_sh.at[pl.ds(r0 + k * CHUNK, CHUNK)])
        rem = rows_per_tile - nfull * CHUNK
        if rem:
            q0 = r0 + nfull * CHUNK
            pltpu.sync_copy(rows0.at[pl.ds(0, rem)], g_sh.at[pl.ds(q0, rem)])
            pltpu.sync_copy(wrow0.at[pl.ds(0, rem)], s_sh.at[pl.ds(q0, rem)])
        plsc.subcore_barrier()

        def scale(rows, wrow, c):
            base = c * CHUNK

            def group_body(g, _):
                w16 = ew_v[pl.ds(base + g * LANES, LANES)]
                for l in range(LANES):
                    i = g * LANES + l
                    w = jnp.broadcast_to(w16[l], (LANES,))
                    vals = [rows[i, pl.ds(j * LANES, LANES)]
                            for j in range(ch // LANES)]
                    scaled = [v * w for v in vals]
                    for j in range(ch // LANES):
                        rows[i, pl.ds(j * LANES, LANES)] = scaled[j]
                    wrow[i, :] = w
                return 0
            lax.fori_loop(0, CHUNK // LANES, group_body, 0)

        def gather(rows, c, gsem):
            pltpu.async_copy(x_hbm.at[cid].at[src_v.at[c]], rows, gsem)

        def gather_wait(rows, c, gsem):
            pltpu.make_async_copy(x_hbm.at[cid].at[src_v.at[c]], rows,
                                  gsem).wait()

        def scatter(rows, wrow, c, ssem, tsem):
            pltpu.async_copy(rows, g_sh.at[dst_v.at[c]], ssem, add=True)

            @pl.when(cid == 0)
            def _():
                pltpu.async_copy(wrow, s_sh.at[dst_v.at[c]], tsem, add=True)

        def scatter_wait(rows, wrow, c, ssem, tsem):
            pltpu.make_async_copy(rows, g_sh.at[dst_v.at[c]], ssem).wait()

            @pl.when(cid == 0)
            def _():
                pltpu.make_async_copy(wrow, s_sh.at[dst_v.at[c]], tsem).wait()

        n_ph = n_chunks // NPHASE
        nsup = n_ph // 2
        for ph in range(NPHASE):
            pltpu.sync_copy(
                idx_hbm.at[1, pl.ds(sid * n_chunks + ph * n_ph, n_ph)], src_v)
            pltpu.sync_copy(
                idx_hbm.at[0, pl.ds(sid * n_chunks + ph * n_ph, n_ph)], dst_v)
            pltpu.sync_copy(
                ew_hbm.at[pl.ds(sid * epw + ph * n_ph * CHUNK,
                                n_ph * CHUNK)], ew_v)

            gather(rows0, 0, gsem0)

            def super_body(k, _):
                c0 = 2 * k
                gather_wait(rows0, c0, gsem0)

                @pl.when(k > 0)
                def _():
                    scatter_wait(rows1, wrow1, c0 - 1, ssem1, tsem1)
                gather(rows1, c0 + 1, gsem1)
                scale(rows0, wrow0, c0)
                scatter(rows0, wrow0, c0, ssem0, tsem0)

                gather_wait(rows1, c0 + 1, gsem1)
                scatter_wait(rows0, wrow0, c0, ssem0, tsem0)

                @pl.when(k < nsup - 1)
                def _():
                    gather(rows0, c0 + 2, gsem0)
                scale(rows1, wrow1, c0 + 1)
                scatter(rows1, wrow1, c0 + 1, ssem1, tsem1)
                return 0
            lax.fori_loop(0, nsup, super_body, 0)
            scatter_wait(rows1, wrow1, n_ph - 1, ssem1, tsem1)

        plsc.subcore_barrier()

        pltpu.sync_copy(g_sh.at[pl.ds(r0, rows_per_tile)],
                        g_out.at[cid, pl.ds(r0, rows_per_tile)])

        @pl.when(cid == 0)
        def _():
            pltpu.sync_copy(s_sh.at[pl.ds(r0, rows_per_tile)],
                            s_out.at[pl.ds(r0, rows_per_tile)])

    return sc_kern(idx_p, ew, x_halves)


def _tc_combine(x, g_part, s_part, A, W2, b):
    V, C = x.shape
    ch = g_part.shape[2]
    F = A.shape[1]
    VB = 1000
    grid = (V // VB,)

    def body(x_ref, g_ref, s_ref, a_ref, w2_ref, b_ref, o_ref):
        s = s_ref[:, 0:1]
        sx = x_ref[...] * s
        o_ref[...] = (
            jnp.dot(sx, a_ref[...], preferred_element_type=jnp.float32)
            + jnp.dot(g_ref[0], w2_ref[0], preferred_element_type=jnp.float32)
            + jnp.dot(g_ref[1], w2_ref[1], preferred_element_type=jnp.float32)
            + s * b_ref[...])

    return pl.pallas_call(
        body,
        grid=grid,
        in_specs=[
            pl.BlockSpec((VB, C), lambda i: (i, 0)),
            pl.BlockSpec((NC, VB, ch), lambda i: (0, i, 0)),
            pl.BlockSpec((VB, LANES), lambda i: (i, 0)),
            pl.BlockSpec((C, F), lambda i: (0, 0)),
            pl.BlockSpec((NC, ch, F), lambda i: (0, 0, 0)),
            pl.BlockSpec((1, F), lambda i: (0, 0)),
        ],
        out_specs=pl.BlockSpec((VB, F), lambda i: (i, 0)),
        out_shape=jax.ShapeDtypeStruct((V, F), jnp.float32),
    )(x, g_part, s_part, A, W2, b.reshape(1, F))


def kernel(x, edge_index, edge_weight, W, b):
    V, C = x.shape
    E = edge_index.shape[1]
    F = W.shape[1]
    ch = C // NC

    epw = -(-E // (NS * 8 * CHUNK)) * (8 * CHUNK)
    e_pad = epw * NS
    n_chunks = epw // CHUNK
    v_pad = -(-V // (NS * 128)) * (NS * 128)

    pad = e_pad - E
    pad_idx = (jnp.arange(pad, dtype=jnp.int32) * 13) % V
    idx_p = jnp.concatenate(
        [edge_index, jnp.broadcast_to(pad_idx[None], (2, pad))],
        axis=1).reshape(2, NS * n_chunks, CHUNK)
    ew_p = jnp.concatenate([edge_weight, jnp.zeros((pad,), jnp.float32)])
    x_halves = jnp.stack([x[:, :ch], x[:, ch:]])

    g_part, s_part = _sc_accumulate(idx_p, ew_p, x_halves, v_pad, n_chunks)

    W1 = W[:C]
    W2 = W[C:]
    w2_halves = jnp.stack([W2[:ch], W2[ch:]])
    return _tc_combine(x, g_part, s_part, W1 - W2, w2_halves, b)


# speedup vs baseline: 10.9005x; 1.1102x over previous
import functools

import jax
import jax.numpy as jnp
from jax import lax
from jax.experimental import pallas as pl
from jax.experimental.pallas import tpu as pltpu
from jax.experimental.pallas import tpu_sc as plsc

NC = 2
NS = 16
NW = NC * NS
CHUNK = 256
LANES = 16
NPHASE = 2


def _sc_accumulate(idx_p, ew, x_halves, v_pad, n_chunks):
    ch = x_halves.shape[2]
    epw = n_chunks * CHUNK
    rows_per_tile = v_pad // NS
    mesh = plsc.VectorSubcoreMesh(core_axis_name="c", subcore_axis_name="s")

    @functools.partial(
        pl.kernel,
        out_type=[
            jax.ShapeDtypeStruct((NC, v_pad, ch), jnp.float32),
            jax.ShapeDtypeStruct((NC, v_pad, LANES), jnp.float32),
        ],
        mesh=mesh,
        compiler_params=pltpu.CompilerParams(use_tc_tiling_on_sc=False),
        scratch_types=[
            pltpu.VMEM((n_chunks // NPHASE, CHUNK), jnp.int32),
            pltpu.VMEM((n_chunks // NPHASE, CHUNK), jnp.int32),
            pltpu.VMEM((epw // NPHASE,), jnp.float32),
            pltpu.VMEM((CHUNK, ch), jnp.float32),
            pltpu.VMEM((CHUNK, ch), jnp.float32),
            pltpu.VMEM((CHUNK, LANES), jnp.float32),
            pltpu.VMEM((CHUNK, LANES), jnp.float32),
            pltpu.VMEM_SHARED((v_pad, ch), jnp.float32),
            pltpu.VMEM_SHARED((v_pad, LANES), jnp.float32),
            pltpu.SemaphoreType.DMA,
            pltpu.SemaphoreType.DMA,
            pltpu.SemaphoreType.DMA,
            pltpu.SemaphoreType.DMA,
            pltpu.SemaphoreType.DMA,
            pltpu.SemaphoreType.DMA,
        ],
    )
    def sc_kern(idx_hbm, ew_hbm, x_hbm, g_out, s_out,
                src_v, dst_v, ew_v, rows0, rows1, wrow0, wrow1, g_sh, s_sh,
                gsem0, gsem1, ssem0, ssem1, tsem0, tsem1):
        cid = lax.axis_index("c")
        sid = lax.axis_index("s")

        zeros16 = jnp.zeros((LANES,), jnp.float32)

        def zero_body(i, _):
            for j in range(ch // LANES):
                rows0[i, pl.ds(j * LANES, LANES)] = zeros16
            wrow0[i, :] = zeros16
            return 0
        lax.fori_loop(0, CHUNK, zero_body, 0)

        r0 = sid * rows_per_tile
        nfull = rows_per_tile // CHUNK
        for k in range(nfull):
            pltpu.sync_copy(rows0, g_sh.at[pl.ds(r0 + k * CHUNK, CHUNK)])
            pltpu.sync_copy(wrow0, s_sh.at[pl.ds(r0 + k * CHUNK, CHUNK)])
        rem = rows_per_tile - nfull * CHUNK
        if rem:
            q0 = r0 + nfull * CHUNK
            pltpu.sync_copy(rows0.at[pl.ds(0, rem)], g_sh.at[pl.ds(q0, rem)])
            pltpu.sync_copy(wrow0.at[pl.ds(0, rem)], s_sh.at[pl.ds(q0, rem)])
        plsc.subcore_barrier()

        def scale(rows, wrow, c, par):
            base = c * CHUNK

            def group_body(g, _):
                w16 = ew_v[pl.ds(base + g * LANES, LANES)]
                for l in range(LANES):
                    i = g * LANES + l
                    w = jnp.broadcast_to(w16[l], (LANES,))
                    vals = [rows[i, pl.ds(j * LANES, LANES)]
                            for j in range(ch // LANES)]
                    scaled = [v * w for v in vals]
                    for j in range(ch // LANES):
                        rows[i, pl.ds(j * LANES, LANES)] = scaled[j]
                return 0
            lax.fori_loop(0, CHUNK // LANES, group_body, 0)

            @pl.when(cid == par)
            def _():
                def w_body(g, _):
                    w16 = ew_v[pl.ds(base + g * LANES, LANES)]
                    for l in range(LANES):
                        wrow[g * LANES + l, :] = jnp.broadcast_to(
                            w16[l], (LANES,))
                    return 0
                lax.fori_loop(0, CHUNK // LANES, w_body, 0)

        def gather(rows, c, gsem):
            pltpu.async_copy(x_hbm.at[cid].at[src_v.at[c]], rows, gsem)

        def gather_wait(rows, c, gsem):
            pltpu.make_async_copy(x_hbm.at[cid].at[src_v.at[c]], rows,
                                  gsem).wait()

        def scatter(rows, wrow, c, ssem, tsem, par):
            pltpu.async_copy(rows, g_sh.at[dst_v.at[c]], ssem, add=True)

            @pl.when(cid == par)
            def _():
                pltpu.async_copy(wrow, s_sh.at[dst_v.at[c]], tsem, add=True)

        def scatter_wait(rows, wrow, c, ssem, tsem, par):
            pltpu.make_async_copy(rows, g_sh.at[dst_v.at[c]], ssem).wait()

            @pl.when(cid == par)
            def _():
                pltpu.make_async_copy(wrow, s_sh.at[dst_v.at[c]], tsem).wait()

        n_ph = n_chunks // NPHASE
        nsup = n_ph // 2
        for ph in range(NPHASE):
            pltpu.sync_copy(
                idx_hbm.at[1, pl.ds(sid * n_chunks + ph * n_ph, n_ph)], src_v)
            pltpu.sync_copy(
                idx_hbm.at[0, pl.ds(sid * n_chunks + ph * n_ph, n_ph)], dst_v)
            pltpu.sync_copy(
                ew_hbm.at[pl.ds(sid * epw + ph * n_ph * CHUNK,
                                n_ph * CHUNK)], ew_v)

            gather(rows0, 0, gsem0)

            def super_body(k, _):
                c0 = 2 * k
                gather_wait(rows0, c0, gsem0)

                @pl.when(k > 0)
                def _():
                    scatter_wait(rows1, wrow1, c0 - 1, ssem1, tsem1, 1)
                gather(rows1, c0 + 1, gsem1)
                scale(rows0, wrow0, c0, 0)
                scatter(rows0, wrow0, c0, ssem0, tsem0, 0)

                gather_wait(rows1, c0 + 1, gsem1)
                scatter_wait(rows0, wrow0, c0, ssem0, tsem0, 0)

                @pl.when(k < nsup - 1)
                def _():
                    gather(rows0, c0 + 2, gsem0)
                scale(rows1, wrow1, c0 + 1, 1)
                scatter(rows1, wrow1, c0 + 1, ssem1, tsem1, 1)
                return 0
            lax.fori_loop(0, nsup, super_body, 0)
            scatter_wait(rows1, wrow1, n_ph - 1, ssem1, tsem1, 1)

        plsc.subcore_barrier()

        pltpu.sync_copy(g_sh.at[pl.ds(r0, rows_per_tile)],
                        g_out.at[cid, pl.ds(r0, rows_per_tile)])
        pltpu.sync_copy(s_sh.at[pl.ds(r0, rows_per_tile)],
                        s_out.at[cid, pl.ds(r0, rows_per_tile)])

    return sc_kern(idx_p, ew, x_halves)


def _tc_premul(x, A, b):
    V, C = x.shape
    F = A.shape[1]
    VB = 1000
    grid = (V // VB,)

    def body(x_ref, a_ref, b_ref, o_ref):
        o_ref[...] = (
            jnp.dot(x_ref[...], a_ref[...], preferred_element_type=jnp.float32)
            + b_ref[...])

    return pl.pallas_call(
        body,
        grid=grid,
        in_specs=[
            pl.BlockSpec((VB, C), lambda i: (i, 0)),
            pl.BlockSpec((C, F), lambda i: (0, 0)),
            pl.BlockSpec((1, F), lambda i: (0, 0)),
        ],
        out_specs=pl.BlockSpec((VB, F), lambda i: (i, 0)),
        out_shape=jax.ShapeDtypeStruct((V, F), jnp.float32),
    )(x, A, b.reshape(1, F))


def _tc_combine(P, g_part, s_part, W2):
    V, F = P.shape
    ch = g_part.shape[2]
    VB = 1000
    grid = (V // VB,)

    def body(p_ref, g_ref, s_ref, w2_ref, o_ref):
        s = s_ref[0, :, 0:1] + s_ref[1, :, 0:1]
        o_ref[...] = (
            s * p_ref[...]
            + jnp.dot(g_ref[0], w2_ref[0], preferred_element_type=jnp.float32)
            + jnp.dot(g_ref[1], w2_ref[1], preferred_element_type=jnp.float32))

    return pl.pallas_call(
        body,
        grid=grid,
        in_specs=[
            pl.BlockSpec((VB, F), lambda i: (i, 0)),
            pl.BlockSpec((NC, VB, ch), lambda i: (0, i, 0)),
            pl.BlockSpec((NC, VB, LANES), lambda i: (0, i, 0)),
            pl.BlockSpec((NC, ch, F), lambda i: (0, 0, 0)),
        ],
        out_specs=pl.BlockSpec((VB, F), lambda i: (i, 0)),
        out_shape=jax.ShapeDtypeStruct((V, F), jnp.float32),
    )(P, g_part, s_part, W2)


def kernel(x, edge_index, edge_weight, W, b):
    V, C = x.shape
    E = edge_index.shape[1]
    F = W.shape[1]
    ch = C // NC

    epw = -(-E // (NS * 8 * CHUNK)) * (8 * CHUNK)
    e_pad = epw * NS
    n_chunks = epw // CHUNK
    v_pad = -(-V // (NS * 128)) * (NS * 128)

    pad = e_pad - E
    pad_idx = (jnp.arange(pad, dtype=jnp.int32) * 13) % V
    idx_p = jnp.concatenate(
        [edge_index, jnp.broadcast_to(pad_idx[None], (2, pad))],
        axis=1).reshape(2, NS * n_chunks, CHUNK)
    ew_p = jnp.concatenate([edge_weight, jnp.zeros((pad,), jnp.float32)])
    x_halves = jnp.stack([x[:, :ch], x[:, ch:]])

    W1 = W[:C]
    W2 = W[C:]
    w2_halves = jnp.stack([W2[:ch], W2[ch:]])
    P = _tc_premul(x, W1 - W2, b)
    g_part, s_part = _sc_accumulate(idx_p, ew_p, x_halves, v_pad, n_chunks)
    return _tc_combine(P, g_part, s_part, w2_halves)
